# baseline (speedup 1.0000x reference)
import functools

import jax
import jax.numpy as jnp
from jax import lax
from jax.experimental import pallas as pl
from jax.experimental.pallas import tpu as pltpu
from jax.experimental.pallas import tpu_sc as plsc

_D = 64
_B = 16384
_MARGIN = 1.0

_NC = 2
_NS = 16
_NW = _NC * _NS
_CB = _B // _NW
_G = 128
_NCHUNK = _CB // _G
_L = 16


def _sqrt16(x):
    xe = x + 1e-30
    i = plsc.bitcast(xe, jnp.int32)
    i = jnp.int32(0x5F3759DF) - lax.shift_right_logical(i, 1)
    y = plsc.bitcast(i, jnp.float32)
    for _ in range(4):
        y = y * (1.5 - 0.5 * xe * y * y)
    return xe * y


_mesh = plsc.VectorSubcoreMesh(core_axis_name="c", subcore_axis_name="s")


@functools.partial(
    pl.kernel,
    mesh=_mesh,
    compiler_params=pltpu.CompilerParams(
        needs_layout_passes=False, use_tc_tiling_on_sc=False),
    out_type=jax.ShapeDtypeStruct((_NW, _L), jnp.float32),
    scratch_types=[
        pltpu.VMEM((_CB,), jnp.int32),
        pltpu.VMEM((_CB,), jnp.int32),
        pltpu.VMEM((_CB,), jnp.int32),
        pltpu.VMEM((_CB,), jnp.int32),
        pltpu.VMEM((_CB,), jnp.int32),
        pltpu.VMEM((_CB,), jnp.int32),
        pltpu.VMEM((_CB, _D), jnp.float32),
        pltpu.VMEM((_CB, _D), jnp.float32),
        pltpu.VMEM((_CB, _D), jnp.float32),
        pltpu.VMEM((_CB,), jnp.float32),
        pltpu.VMEM((_CB,), jnp.float32),
        pltpu.VMEM((_L,), jnp.float32),
        pltpu.SemaphoreType.DMA,
    ],
)
def _transe_sc(ph, pr, pt, nh, nr, nt, ent, rel, out,
               ph_v, pr_v, pt_v, nh_v, nr_v, nt_v,
               h_v, r_v, t_v, ps_v, ns_v, ob_v, sem):
    wid = lax.axis_index("s") * _NC + lax.axis_index("c")
    base = wid * _CB

    for src, dst in ((ph, ph_v), (pr, pr_v), (pt, pt_v),
                     (nh, nh_v), (nr, nr_v), (nt, nt_v)):
        pltpu.sync_copy(src.at[pl.ds(base, _CB)], dst)

    def gather_side(hi, ri, ti):
        cps = []
        for j in range(_NCHUNK):
            sl = pl.ds(j * _G, _G)
            cps.append(pltpu.async_copy(ent.at[hi.at[sl]], h_v.at[sl], sem))
            cps.append(pltpu.async_copy(rel.at[ri.at[sl]], r_v.at[sl], sem))
            cps.append(pltpu.async_copy(ent.at[ti.at[sl]], t_v.at[sl], sem))
        for c in cps:
            c.wait()

    iot = lax.iota(jnp.int32, _L)

    def compute_side(sums_ref):
        def grpw(g, carry):
            vec = jnp.zeros((_L,), jnp.float32)
            for r in range(_L):
                i = g * _L + r
                acc = jnp.zeros((_L,), jnp.float32)
                for j in range(_D // _L):
                    sl = pl.ds(j * _L, _L)
                    d = h_v[i, sl] + r_v[i, sl] - t_v[i, sl]
                    acc = acc + d * d
                s = lax.reduce_sum(acc, axes=(0,))
                vec = jnp.where(iot == r, s, vec)
            sums_ref[pl.ds(g * _L, _L)] = vec
            return carry
        lax.fori_loop(0, _CB // _L, grpw, 0)

    gather_side(ph_v, pr_v, pt_v)
    compute_side(ps_v)
    gather_side(nh_v, nr_v, nt_v)
    compute_side(ns_v)

    def grp(g, acc):
        ps = ps_v[pl.ds(g * _L, _L)]
        ns = ns_v[pl.ds(g * _L, _L)]
        sp = _sqrt16(ps)
        sn = _sqrt16(ns)
        return acc + jnp.maximum(_MARGIN + sp - sn, 0.0)

    acc16 = lax.fori_loop(0, _CB // _L, grp, jnp.zeros((_L,), jnp.float32))
    ob_v[:] = acc16 * (1.0 / _B)
    pltpu.sync_copy(ob_v, out.at[wid])


def kernel(pos_h, pos_r, pos_t, neg_h, neg_r, neg_t,
           entity_embedding, relation_embedding):
    idx = [a.astype(jnp.int32) for a in
           (pos_h, pos_r, pos_t, neg_h, neg_r, neg_t)]
    partials = _transe_sc(*idx, entity_embedding, relation_embedding)
    return jnp.sum(partials)


# speedup vs baseline: 1.2281x; 1.2281x over previous
import functools

import jax
import jax.numpy as jnp
from jax import lax
from jax.experimental import pallas as pl
from jax.experimental.pallas import tpu as pltpu
from jax.experimental.pallas import tpu_sc as plsc

_E = 1000000
_R = 1000
_D = 64
_B = 16384
_MARGIN = 1.0

_NC = 2
_NS = 16
_NW = _NC * _NS
_CB = _B // _NW
_H = 256
_SUB = 16
_L = 16


def _sqrt16(x):
    xe = x + 1e-30
    i = plsc.bitcast(xe, jnp.int32)
    i = jnp.int32(0x5F3759DF) - lax.shift_right_logical(i, 1)
    y = plsc.bitcast(i, jnp.float32)
    for _ in range(4):
        y = y * (1.5 - 0.5 * xe * y * y)
    return xe * y


_mesh = plsc.VectorSubcoreMesh(core_axis_name="c", subcore_axis_name="s")


@functools.partial(
    pl.kernel,
    mesh=_mesh,
    compiler_params=pltpu.CompilerParams(
        needs_layout_passes=False, use_tc_tiling_on_sc=True),
    out_type=jax.ShapeDtypeStruct((_NW, _L), jnp.float32),
    scratch_types=[
        pltpu.VMEM((_CB,), jnp.int32),
        pltpu.VMEM((_CB,), jnp.int32),
        pltpu.VMEM((_CB,), jnp.int32),
        pltpu.VMEM((_H // 2, 128), jnp.float32),
        pltpu.VMEM((_H // 2, 128), jnp.float32),
        pltpu.VMEM((_H // 2, 128), jnp.float32),
        pltpu.VMEM((_CB,), jnp.float32),
        pltpu.VMEM((_CB,), jnp.float32),
        pltpu.VMEM((_L,), jnp.float32),
        pltpu.SemaphoreType.DMA,
    ],
)
def _transe_sc(ph, pr, pt, nh, nr, nt, ent, rel, out,
               hs_s, rs_s, ts_s,
               h_v, r_v, t_v, ps_v, ns_v, ob_v, sem):
    wid = lax.axis_index("s") * _NC + lax.axis_index("c")
    base = wid * _CB

    iot = lax.iota(jnp.int32, _L)

    def side(h_idx, r_idx, t_idx, sums_v):
        for src, dst in ((h_idx, hs_s), (r_idx, rs_s), (t_idx, ts_s)):
            pltpu.sync_copy(src.at[pl.ds(base, _CB)], dst)

        for half in range(_CB // _H):
            row_base = half * _H

            def sub(s, _):
                isl = pl.ds(pl.multiple_of(row_base + s * _SUB, 8), _SUB)
                hvec = hs_s[isl]
                rvec = rs_s[isl]
                tvec = ts_s[isl]
                cps = []
                for r in range(_SUB):
                    vrow = s * (_SUB // 2) + r // 2
                    dsl = pl.ds((r % 2) * _D, _D)
                    cps.append(pltpu.async_copy(
                        ent.at[hvec[r]], h_v.at[vrow, dsl], sem))
                    cps.append(pltpu.async_copy(
                        rel.at[rvec[r]], r_v.at[vrow, dsl], sem))
                    cps.append(pltpu.async_copy(
                        ent.at[tvec[r]], t_v.at[vrow, dsl], sem))
                for c in cps:
                    c.wait()
                return _
            lax.fori_loop(0, _H // _SUB, sub, 0)

            def grp(g, _):
                glo = pl.ds(pl.multiple_of(row_base + g * _L, 8), _L)
                k16 = iot + g * _L
                krow = lax.shift_right_logical(k16, 1)
                kcol0 = lax.bitwise_and(k16, jnp.full((_L,), 1, jnp.int32)) * _D
                acc = jnp.zeros((_L,), jnp.float32)
                for col in range(_D):
                    kc = kcol0 + col
                    hv = plsc.load_gather(h_v, [krow, kc])
                    rv = plsc.load_gather(r_v, [krow, kc])
                    tv = plsc.load_gather(t_v, [krow, kc])
                    d = hv + rv - tv
                    acc = acc + d * d
                sums_v[glo] = acc
                return _
            lax.fori_loop(0, _H // _L, grp, 0)

    side(ph, pr, pt, ps_v)
    side(nh, nr, nt, ns_v)

    def fin(g, acc):
        sl = pl.ds(pl.multiple_of(g * _L, 8), _L)
        sp = _sqrt16(ps_v[sl])
        sn = _sqrt16(ns_v[sl])
        return acc + jnp.maximum(_MARGIN + sp - sn, 0.0)

    acc16 = lax.fori_loop(0, _CB // _L, fin, jnp.zeros((_L,), jnp.float32))
    ob_v[:] = acc16 * (1.0 / _B)
    pltpu.sync_copy(ob_v, out.at[wid])


def kernel(pos_h, pos_r, pos_t, neg_h, neg_r, neg_t,
           entity_embedding, relation_embedding):
    idx = [a.astype(jnp.int32) for a in
           (pos_h, pos_r, pos_t, neg_h, neg_r, neg_t)]
    partials = _transe_sc(*idx, entity_embedding, relation_embedding)
    return jnp.sum(partials)


# speedup vs baseline: 1.3005x; 1.0589x over previous
import functools

import jax
import jax.numpy as jnp
from jax import lax
from jax.experimental import pallas as pl
from jax.experimental.pallas import tpu as pltpu
from jax.experimental.pallas import tpu_sc as plsc

_E = 1000000
_R = 1000
_D = 64
_B = 16384
_MARGIN = 1.0

_NC = 2
_NS = 16
_NW = _NC * _NS
_CB = _B // _NW
_H = 256
_SUB = 16
_L = 16


def _sqrt16(x):
    xe = x + 1e-30
    i = plsc.bitcast(xe, jnp.int32)
    i = jnp.int32(0x5F3759DF) - lax.shift_right_logical(i, 1)
    y = plsc.bitcast(i, jnp.float32)
    for _ in range(4):
        y = y * (1.5 - 0.5 * xe * y * y)
    return xe * y


_mesh = plsc.VectorSubcoreMesh(core_axis_name="c", subcore_axis_name="s")


@functools.partial(
    pl.kernel,
    mesh=_mesh,
    compiler_params=pltpu.CompilerParams(
        needs_layout_passes=False, use_tc_tiling_on_sc=True),
    out_type=jax.ShapeDtypeStruct((_NW, _L), jnp.float32),
    scratch_types=[
        pltpu.VMEM((_CB,), jnp.int32),
        pltpu.VMEM((_CB,), jnp.int32),
        pltpu.VMEM((_CB,), jnp.int32),
        pltpu.VMEM((_H // 2, 128), jnp.float32),
        pltpu.VMEM((_H // 2, 128), jnp.float32),
        pltpu.VMEM((_H // 2, 128), jnp.float32),
        pltpu.VMEM((_CB,), jnp.float32),
        pltpu.VMEM((_CB,), jnp.float32),
        pltpu.VMEM((_L,), jnp.float32),
        pltpu.SemaphoreType.DMA,
    ],
)
def _transe_sc(ph, pr, pt, nh, nr, nt, ent, rel, dmy, out,
               hs_s, rs_s, ts_s,
               h_v, r_v, t_v, ps_v, ns_v, ob_v, sem):
    wid = lax.axis_index("s") * _NC + lax.axis_index("c")
    base = wid * _CB

    iot = lax.iota(jnp.int32, _L)

    def side(h_idx, r_idx, t_idx, sums_v):
        for src, dst in ((h_idx, hs_s), (r_idx, rs_s), (t_idx, ts_s)):
            pltpu.sync_copy(src.at[pl.ds(base, _CB)], dst)

        def half_body(half, _):
            row_base = half * _H

            def sub(s, _):
                isl = pl.ds(pl.multiple_of(row_base + s * _SUB, 8), _SUB)
                hvec = hs_s[isl]
                rvec = rs_s[isl]
                tvec = ts_s[isl]
                for r in range(_SUB):
                    vrow = s * (_SUB // 2) + r // 2
                    dsl = pl.ds((r % 2) * _D, _D)
                    pltpu.async_copy(ent.at[hvec[r]], h_v.at[vrow, dsl], sem)
                    pltpu.async_copy(rel.at[rvec[r]], r_v.at[vrow, dsl], sem)
                    pltpu.async_copy(ent.at[tvec[r]], t_v.at[vrow, dsl], sem)
                vsl = pl.ds(pl.multiple_of(s * (_SUB // 2), 8), _SUB // 2)
                for buf in (h_v, r_v, t_v):
                    pltpu.make_async_copy(dmy, buf.at[vsl], sem).wait()
                return _
            lax.fori_loop(0, _H // _SUB, sub, 0)

            def grp(g, _):
                glo = pl.ds(pl.multiple_of(row_base + g * _L, 8), _L)
                k16 = iot + g * _L
                krow = lax.shift_right_logical(k16, 1)
                kcol0 = lax.bitwise_and(k16, jnp.full((_L,), 1, jnp.int32)) * _D
                acc = jnp.zeros((_L,), jnp.float32)

                def colblk(cb, acc):
                    kc0 = kcol0 + cb * 8
                    for col in range(8):
                        kc = kc0 + col
                        hv = plsc.load_gather(h_v, [krow, kc])
                        rv = plsc.load_gather(r_v, [krow, kc])
                        tv = plsc.load_gather(t_v, [krow, kc])
                        d = hv + rv - tv
                        acc = acc + d * d
                    return acc
                acc = lax.fori_loop(0, _D // 8, colblk, acc)
                sums_v[glo] = acc
                return _
            lax.fori_loop(0, _H // _L, grp, 0)
            return _
        lax.fori_loop(0, _CB // _H, half_body, 0)

    side(ph, pr, pt, ps_v)
    side(nh, nr, nt, ns_v)

    def fin(g, acc):
        sl = pl.ds(pl.multiple_of(g * _L, 8), _L)
        sp = _sqrt16(ps_v[sl])
        sn = _sqrt16(ns_v[sl])
        return acc + jnp.maximum(_MARGIN + sp - sn, 0.0)

    acc16 = lax.fori_loop(0, _CB // _L, fin, jnp.zeros((_L,), jnp.float32))
    ob_v[:] = acc16 * (1.0 / _B)
    pltpu.sync_copy(ob_v, out.at[wid])


def kernel(pos_h, pos_r, pos_t, neg_h, neg_r, neg_t,
           entity_embedding, relation_embedding):
    idx = [a.astype(jnp.int32) for a in
           (pos_h, pos_r, pos_t, neg_h, neg_r, neg_t)]
    dmy = jnp.zeros((_SUB // 2, 128), jnp.float32)
    partials = _transe_sc(*idx, entity_embedding, relation_embedding, dmy)
    return jnp.sum(partials)


# speedup vs baseline: 1.3630x; 1.0480x over previous
import functools

import jax
import jax.numpy as jnp
from jax import lax
from jax.experimental import pallas as pl
from jax.experimental.pallas import tpu as pltpu
from jax.experimental.pallas import tpu_sc as plsc

_E = 1000000
_R = 1000
_D = 64
_B = 16384
_MARGIN = 1.0

_NC = 2
_NS = 16
_NW = _NC * _NS
_CB = _B // _NW
_TR = 2 * _CB
_NQ = 4
_QR = _TR // _NQ
_SUB = 16
_NSB = _QR // _SUB
_L = 16


def _sqrt16(x):
    xe = x + 1e-30
    i = plsc.bitcast(xe, jnp.int32)
    i = jnp.int32(0x5F3759DF) - lax.shift_right_logical(i, 1)
    y = plsc.bitcast(i, jnp.float32)
    for _ in range(4):
        y = y * (1.5 - 0.5 * xe * y * y)
    return xe * y


_mesh = plsc.VectorSubcoreMesh(core_axis_name="c", subcore_axis_name="s")


@functools.partial(
    pl.kernel,
    mesh=_mesh,
    compiler_params=pltpu.CompilerParams(
        needs_layout_passes=False, use_tc_tiling_on_sc=True),
    out_type=jax.ShapeDtypeStruct((_NW, _L), jnp.float32),
    scratch_types=[
        pltpu.VMEM((_TR,), jnp.float32),
        pltpu.VMEM((_TR,), jnp.float32),
        pltpu.VMEM((_TR,), jnp.float32),
        pltpu.VMEM((2, _QR // 2, 128), jnp.float32),
        pltpu.VMEM((2, _QR // 2, 128), jnp.float32),
        pltpu.VMEM((2, _QR // 2, 128), jnp.float32),
        pltpu.VMEM((_TR,), jnp.float32),
        pltpu.VMEM((_L,), jnp.float32),
        pltpu.SemaphoreType.DMA,
        pltpu.SemaphoreType.DMA,
    ],
)
def _transe_sc(ph, pr, pt, nh, nr, nt, ent, rel, dmy, out,
               hs_s, rs_s, ts_s, h_v, r_v, t_v, sums_v, ob_v,
               sem0, sem1):
    wid = lax.axis_index("s") * _NC + lax.axis_index("c")
    base = wid * _CB

    pltpu.sync_copy(ph.at[pl.ds(base, _CB)], hs_s.at[pl.ds(0, _CB)])
    pltpu.sync_copy(pr.at[pl.ds(base, _CB)], rs_s.at[pl.ds(0, _CB)])
    pltpu.sync_copy(pt.at[pl.ds(base, _CB)], ts_s.at[pl.ds(0, _CB)])
    pltpu.sync_copy(nh.at[pl.ds(base, _CB)], hs_s.at[pl.ds(_CB, _CB)])
    pltpu.sync_copy(nr.at[pl.ds(base, _CB)], rs_s.at[pl.ds(_CB, _CB)])
    pltpu.sync_copy(nt.at[pl.ds(base, _CB)], ts_s.at[pl.ds(_CB, _CB)])

    iot = lax.iota(jnp.int32, _L)

    def fire_batch(rbase, s, par, sem):
        isl = pl.ds(pl.multiple_of(rbase + s * _SUB, 8), _SUB)
        hvec = plsc.bitcast(hs_s[isl], jnp.int32)
        rvec = plsc.bitcast(rs_s[isl], jnp.int32)
        tvec = plsc.bitcast(ts_s[isl], jnp.int32)
        for r in range(_SUB):
            vrow = s * (_SUB // 2) + r // 2
            dsl = pl.ds((r % 2) * _D, _D)
            pltpu.async_copy(ent.at[hvec[r]], h_v.at[par, vrow, dsl], sem)
            pltpu.async_copy(rel.at[rvec[r]], r_v.at[par, vrow, dsl], sem)
            pltpu.async_copy(ent.at[tvec[r]], t_v.at[par, vrow, dsl], sem)

    def drain_batch(sem):
        for buf in (h_v, r_v, t_v):
            pltpu.make_async_copy(
                dmy, buf.at[0, pl.ds(0, _SUB // 2)], sem).wait()

    def compute_batch(rbase, s, par):
        glo = pl.ds(pl.multiple_of(rbase + s * _L, 8), _L)
        k16 = iot + s * _L
        par16 = jnp.full((_L,), par, jnp.int32)
        krow = lax.shift_right_logical(k16, 1)
        kcol0 = lax.bitwise_and(k16, jnp.full((_L,), 1, jnp.int32)) * _D
        acc = jnp.zeros((_L,), jnp.float32)

        def colblk(cb, acc):
            kc0 = kcol0 + cb * 8
            for col in range(8):
                kc = kc0 + col
                hv = plsc.load_gather(h_v, [par16, krow, kc])
                rv = plsc.load_gather(r_v, [par16, krow, kc])
                tv = plsc.load_gather(t_v, [par16, krow, kc])
                d = hv + rv - tv
                acc = acc + d * d
            return acc
        acc = lax.fori_loop(0, _D // 8, colblk, acc)
        sums_v[glo] = acc

    def pro(s, carry):
        fire_batch(0, s, 0, sem0)

        @pl.when(s > 0)
        def _drain():
            drain_batch(sem0)
        return carry
    lax.fori_loop(0, _NSB, pro, 0)
    drain_batch(sem0)

    def quarter(q, par, semn):
        nbase = jnp.minimum((q + 1) * _QR, (_NQ - 1) * _QR)
        qbase = q * _QR
        do_fire = q < (_NQ - 1)

        def it(s, carry):
            @pl.when(do_fire)
            def _fire():
                fire_batch(nbase, s, 1 - par, semn)

            @pl.when(jnp.logical_and(do_fire, s > 0))
            def _drain():
                drain_batch(semn)
            compute_batch(qbase, s, par)
            return carry
        lax.fori_loop(0, _NSB, it, 0)

        @pl.when(do_fire)
        def _final_drain():
            drain_batch(semn)

    def super_it(i, carry):
        quarter(2 * i, 0, sem1)
        quarter(2 * i + 1, 1, sem0)
        return carry
    lax.fori_loop(0, _NQ // 2, super_it, 0)

    def fin(g, acc):
        psl = pl.ds(pl.multiple_of(g * _L, 8), _L)
        nsl = pl.ds(pl.multiple_of(_CB + g * _L, 8), _L)
        sp = _sqrt16(sums_v[psl])
        sn = _sqrt16(sums_v[nsl])
        return acc + jnp.maximum(_MARGIN + sp - sn, 0.0)

    acc16 = lax.fori_loop(0, _CB // _L, fin, jnp.zeros((_L,), jnp.float32))
    ob_v[:] = acc16 * (1.0 / _B)
    pltpu.sync_copy(ob_v, out.at[wid])


def kernel(pos_h, pos_r, pos_t, neg_h, neg_r, neg_t,
           entity_embedding, relation_embedding):
    idx = [lax.bitcast_convert_type(a.astype(jnp.int32), jnp.float32)
           for a in (pos_h, pos_r, pos_t, neg_h, neg_r, neg_t)]
    dmy = jnp.zeros((_SUB // 2, 128), jnp.float32)
    partials = _transe_sc(*idx, entity_embedding, relation_embedding, dmy)
    return jnp.sum(partials)


# speedup vs baseline: 1.3770x; 1.0103x over previous
import functools

import jax
import jax.numpy as jnp
from jax import lax
from jax.experimental import pallas as pl
from jax.experimental.pallas import tpu as pltpu
from jax.experimental.pallas import tpu_sc as plsc

_E = 1000000
_R = 1000
_D = 64
_B = 16384
_MARGIN = 1.0

_NC = 2
_NS = 16
_NW = _NC * _NS
_CB = _B // _NW
_TR = 2 * _CB
_NQ = 8
_QR = _TR // _NQ
_SUB = 32
_NSB = _QR // _SUB
_L = 16


def _sqrt16(x):
    xe = x + 1e-30
    i = plsc.bitcast(xe, jnp.int32)
    i = jnp.int32(0x5F3759DF) - lax.shift_right_logical(i, 1)
    y = plsc.bitcast(i, jnp.float32)
    for _ in range(4):
        y = y * (1.5 - 0.5 * xe * y * y)
    return xe * y


_mesh = plsc.VectorSubcoreMesh(core_axis_name="c", subcore_axis_name="s")


@functools.partial(
    pl.kernel,
    mesh=_mesh,
    compiler_params=pltpu.CompilerParams(
        needs_layout_passes=False, use_tc_tiling_on_sc=True),
    out_type=jax.ShapeDtypeStruct((_NW, _L), jnp.float32),
    scratch_types=[
        pltpu.VMEM((_TR,), jnp.float32),
        pltpu.VMEM((_TR,), jnp.float32),
        pltpu.VMEM((_TR,), jnp.float32),
        pltpu.VMEM((2, _QR // 2, 128), jnp.float32),
        pltpu.VMEM((2, _QR, 128), jnp.float32),
        pltpu.VMEM((2, _QR // 2, 128), jnp.float32),
        pltpu.VMEM((_TR,), jnp.int32),
        pltpu.VMEM((_TR,), jnp.float32),
        pltpu.VMEM((_L,), jnp.float32),
        pltpu.SemaphoreType.DMA,
        pltpu.SemaphoreType.DMA,
    ],
)
def _transe_sc(ph, pr, pt, nh, nr, nt, ent, rel, dmy, out,
               hs_s, rs_s, ts_s, h_v, r_v, t_v, ri_v, sums_v, ob_v,
               sem0, sem1):
    wid = lax.axis_index("s") * _NC + lax.axis_index("c")
    base = wid * _CB

    pltpu.sync_copy(ph.at[pl.ds(base, _CB)], hs_s.at[pl.ds(0, _CB)])
    pltpu.sync_copy(pr.at[pl.ds(base, _CB)], rs_s.at[pl.ds(0, _CB)])
    pltpu.sync_copy(pt.at[pl.ds(base, _CB)], ts_s.at[pl.ds(0, _CB)])
    pltpu.sync_copy(nh.at[pl.ds(base, _CB)], hs_s.at[pl.ds(_CB, _CB)])
    pltpu.sync_copy(nr.at[pl.ds(base, _CB)], rs_s.at[pl.ds(_CB, _CB)])
    pltpu.sync_copy(nt.at[pl.ds(base, _CB)], ts_s.at[pl.ds(_CB, _CB)])

    iot = lax.iota(jnp.int32, _L)

    def cvt(g, carry):
        sl = pl.ds(pl.multiple_of(g * _L, 8), _L)
        ri_v[sl] = plsc.bitcast(rs_s[sl], jnp.int32)
        return carry
    lax.fori_loop(0, _TR // _L, cvt, 0)

    def fire_batch(rbase, s, par, sem):
        iosl = pl.ds(pl.multiple_of(rbase + s * _SUB, 8), _SUB)
        pltpu.async_copy(rel.at[ri_v.at[iosl]],
                         r_v.at[par, pl.ds(s * _SUB, _SUB)], sem)
        for b in range(_SUB // 16):
            isl = pl.ds(pl.multiple_of(rbase + s * _SUB + b * 16, 8), 16)
            hvec = plsc.bitcast(hs_s[isl], jnp.int32)
            tvec = plsc.bitcast(ts_s[isl], jnp.int32)
            for r in range(16):
                vrow = s * (_SUB // 2) + b * 8 + r // 2
                dsl = pl.ds((r % 2) * _D, _D)
                pltpu.async_copy(ent.at[hvec[r]], h_v.at[par, vrow, dsl], sem)
                pltpu.async_copy(ent.at[tvec[r]], t_v.at[par, vrow, dsl], sem)

    def drain_batch(sem):
        for _n in range(4):
            pltpu.make_async_copy(
                dmy, h_v.at[0, pl.ds(0, _SUB // 2)], sem).wait()

    def compute_group(rbase, g, par):
        glo = pl.ds(pl.multiple_of(rbase + g * _L, 8), _L)
        k16 = iot + g * _L
        par16 = jnp.full((_L,), par, jnp.int32)
        krow = lax.shift_right_logical(k16, 1)
        kcol0 = lax.bitwise_and(k16, jnp.full((_L,), 1, jnp.int32)) * _D
        acc = jnp.zeros((_L,), jnp.float32)

        def colblk(cb, acc):
            kc0 = kcol0 + cb * 8
            for col in range(8):
                kc = kc0 + col
                kcr = kc - kcol0
                hv = plsc.load_gather(h_v, [par16, krow, kc])
                rv = plsc.load_gather(r_v, [par16, k16, kcr])
                tv = plsc.load_gather(t_v, [par16, krow, kc])
                d = hv + rv - tv
                acc = acc + d * d
            return acc
        acc = lax.fori_loop(0, _D // 8, colblk, acc)
        sums_v[glo] = acc

    def compute_batch(rbase, s, par):
        for g2 in range(_SUB // _L):
            compute_group(rbase, s * (_SUB // _L) + g2, par)

    def pro(s, carry):
        fire_batch(0, s, 0, sem0)

        @pl.when(s > 0)
        def _drain():
            drain_batch(sem0)
        return carry
    lax.fori_loop(0, _NSB, pro, 0)
    drain_batch(sem0)

    def quarter(q, par, semn):
        nbase = jnp.minimum((q + 1) * _QR, (_NQ - 1) * _QR)
        qbase = q * _QR
        do_fire = q < (_NQ - 1)

        def it(s, carry):
            @pl.when(do_fire)
            def _fire():
                fire_batch(nbase, s, 1 - par, semn)

            @pl.when(jnp.logical_and(do_fire, s > 0))
            def _drain():
                drain_batch(semn)
            compute_batch(qbase, s, par)
            return carry
        lax.fori_loop(0, _NSB, it, 0)

        @pl.when(do_fire)
        def _final_drain():
            drain_batch(semn)

    def super_it(i, carry):
        quarter(2 * i, 0, sem1)
        quarter(2 * i + 1, 1, sem0)
        return carry
    lax.fori_loop(0, _NQ // 2, super_it, 0)

    def fin(g, acc):
        psl = pl.ds(pl.multiple_of(g * _L, 8), _L)
        nsl = pl.ds(pl.multiple_of(_CB + g * _L, 8), _L)
        sp = _sqrt16(sums_v[psl])
        sn = _sqrt16(sums_v[nsl])
        return acc + jnp.maximum(_MARGIN + sp - sn, 0.0)

    acc16 = lax.fori_loop(0, _CB // _L, fin, jnp.zeros((_L,), jnp.float32))
    ob_v[:] = acc16 * (1.0 / _B)
    pltpu.sync_copy(ob_v, out.at[wid])


def kernel(pos_h, pos_r, pos_t, neg_h, neg_r, neg_t,
           entity_embedding, relation_embedding):
    idx = [lax.bitcast_convert_type(a.astype(jnp.int32), jnp.float32)
           for a in (pos_h, pos_r, pos_t, neg_h, neg_r, neg_t)]
    dmy = jnp.zeros((_SUB // 2, 128), jnp.float32)
    relp = jnp.pad(relation_embedding, ((0, 0), (0, 128 - _D)))
    partials = _transe_sc(*idx, entity_embedding, relp, dmy)
    return jnp.sum(partials)
